# Initial kernel scaffold; baseline (speedup 1.0000x reference)
import jax
import jax.numpy as jnp
from jax.experimental import pallas as pl


def kernel(new_token_ids, new_emb_weight):
    raise NotImplementedError("write your pallas kernel here")


# speedup vs baseline: 6.8564x; 6.8564x over previous
import functools

import jax
import jax.numpy as jnp
from jax import lax
from jax.experimental import pallas as pl
from jax.experimental.pallas import tpu as pltpu
from jax.experimental.pallas import tpu_sc as plsc

D_MODEL = 128


@functools.cache
def _make_gather(num_rows: int, d: int, total: int):
    info = plsc.get_sparse_core_info()
    nw = info.num_cores * info.num_subcores
    assert total % nw == 0
    b_per_w = total // nw
    chunk = 256
    assert b_per_w % chunk == 0
    n_chunks = b_per_w // chunk
    mesh = plsc.VectorSubcoreMesh(core_axis_name="c", subcore_axis_name="s")

    @functools.partial(
        pl.kernel,
        mesh=mesh,
        out_type=jax.ShapeDtypeStruct((total, d), jnp.float32),
        scratch_types=[
            pltpu.VMEM((chunk,), jnp.int32),
            pltpu.VMEM((chunk, d), jnp.float32),
            pltpu.SemaphoreType.DMA,
        ],
    )
    def gather_kernel(table_hbm, idx_hbm, out_hbm, idx_v, rows_v, sem):
        wid = lax.axis_index("s") * info.num_cores + lax.axis_index("c")
        base = wid * b_per_w

        def body(i, carry):
            off = base + i * chunk
            pltpu.sync_copy(idx_hbm.at[pl.ds(off, chunk)], idx_v)
            pltpu.async_copy(table_hbm.at[idx_v], rows_v, sem).wait()
            pltpu.sync_copy(rows_v, out_hbm.at[pl.ds(off, chunk)])
            return carry

        lax.fori_loop(0, n_chunks, body, 0)

    return gather_kernel


def kernel(new_token_ids, new_emb_weight):
    b, h = new_token_ids.shape
    v, d = new_emb_weight.shape
    idx = new_token_ids.reshape(-1).astype(jnp.int32)
    out = _make_gather(v, d, b * h)(new_emb_weight, idx)
    return out.reshape(b, h, d)


# speedup vs baseline: 9.1743x; 1.3381x over previous
import functools

import jax
import jax.numpy as jnp
from jax import lax
from jax.experimental import pallas as pl
from jax.experimental.pallas import tpu as pltpu
from jax.experimental.pallas import tpu_sc as plsc

D_MODEL = 128


@functools.cache
def _make_gather(num_rows: int, d: int, total: int, chunk: int):
    info = plsc.get_sparse_core_info()
    nw = info.num_cores * info.num_subcores
    assert total % nw == 0
    b_per_w = total // nw
    assert b_per_w % chunk == 0
    n_chunks = b_per_w // chunk
    assert n_chunks % 2 == 0
    mesh = plsc.VectorSubcoreMesh(core_axis_name="c", subcore_axis_name="s")

    @functools.partial(
        pl.kernel,
        mesh=mesh,
        out_type=jax.ShapeDtypeStruct((total, d), jnp.float32),
        scratch_types=[
            pltpu.VMEM((chunk,), jnp.int32),
            pltpu.VMEM((chunk,), jnp.int32),
            pltpu.VMEM((chunk, d), jnp.float32),
            pltpu.VMEM((chunk, d), jnp.float32),
            pltpu.SemaphoreType.DMA,
            pltpu.SemaphoreType.DMA,
            pltpu.SemaphoreType.DMA,
            pltpu.SemaphoreType.DMA,
            pltpu.SemaphoreType.DMA,
        ],
    )
    def gather_kernel(table_hbm, idx_hbm, out_hbm,
                      idx0, idx1, rows0, rows1,
                      sem_g, st0, st1, si0, si1):
        idx_v = (idx0, idx1)
        rows_v = (rows0, rows1)
        st = (st0, st1)
        si = (si0, si1)
        wid = lax.axis_index("s") * info.num_cores + lax.axis_index("c")
        base = wid * b_per_w

        pltpu.async_copy(idx_hbm.at[pl.ds(base, chunk)], idx0, si0)
        pltpu.async_copy(idx_hbm.at[pl.ds(base + chunk, chunk)], idx1, si1)

        def pair_body(j, carry):
            for k in range(2):
                i = 2 * j + k
                ib, rb, sst, sidx = idx_v[k], rows_v[k], st[k], si[k]
                off = base + i * chunk
                pltpu.make_async_copy(
                    idx_hbm.at[pl.ds(base, chunk)], ib, sidx).wait()

                @pl.when(j > 0)
                def _wait_store():
                    pltpu.make_async_copy(
                        rb, out_hbm.at[pl.ds(base, chunk)], sst).wait()

                pltpu.async_copy(table_hbm.at[ib], rb, sem_g).wait()

                @pl.when(i + 2 < n_chunks)
                def _prefetch_idx():
                    pltpu.async_copy(
                        idx_hbm.at[pl.ds(off + 2 * chunk, chunk)], ib, sidx)

                pltpu.async_copy(rb, out_hbm.at[pl.ds(off, chunk)], sst)
            return carry

        lax.fori_loop(0, n_chunks // 2, pair_body, 0)

        pltpu.make_async_copy(rows0, out_hbm.at[pl.ds(base, chunk)], st0).wait()
        pltpu.make_async_copy(rows1, out_hbm.at[pl.ds(base, chunk)], st1).wait()

    return gather_kernel


def kernel(new_token_ids, new_emb_weight):
    b, h = new_token_ids.shape
    v, d = new_emb_weight.shape
    idx = new_token_ids.reshape(-1).astype(jnp.int32)
    out = _make_gather(v, d, b * h, 256)(new_emb_weight, idx)
    return out.reshape(b, h, d)
